# Initial kernel scaffold; baseline (speedup 1.0000x reference)
import jax
import jax.numpy as jnp
from jax.experimental import pallas as pl


def kernel(values, t, shape):
    raise NotImplementedError("write your pallas kernel here")


# speedup vs baseline: 4.6253x; 4.6253x over previous
import functools

import jax
import jax.numpy as jnp
from jax import lax
from jax.experimental import pallas as pl
from jax.experimental.pallas import tpu as pltpu
from jax.experimental.pallas import tpu_sc as plsc

_NC = 2
_NS = 16
_NW = _NC * _NS
_L = 16


@functools.lru_cache(maxsize=None)
def _make_gather(batch: int, table_padded: int):
    assert batch % (_NW * _L) == 0
    b_per_w = batch // _NW

    mesh = plsc.VectorSubcoreMesh(core_axis_name="c", subcore_axis_name="s")

    @functools.partial(
        pl.kernel,
        out_type=jax.ShapeDtypeStruct((batch,), jnp.float32),
        mesh=mesh,
        scratch_types=[
            pltpu.VMEM((table_padded,), jnp.float32),
            pltpu.VMEM((b_per_w,), jnp.int32),
            pltpu.VMEM((b_per_w,), jnp.float32),
        ],
        compiler_params=pltpu.CompilerParams(needs_layout_passes=False),
    )
    def gather_kernel(values_hbm, t_hbm, out_hbm, tab_v, idx_v, out_v):
        wid = lax.axis_index("s") * _NC + lax.axis_index("c")
        base = wid * b_per_w
        pltpu.sync_copy(values_hbm, tab_v)
        pltpu.sync_copy(t_hbm.at[pl.ds(base, b_per_w)], idx_v)

        def step(i, carry):
            off = i * _L
            idx16 = idx_v[pl.ds(off, _L)]
            out_v[pl.ds(off, _L)] = plsc.load_gather(tab_v, [idx16])
            return carry

        lax.fori_loop(0, b_per_w // _L, step, 0)
        pltpu.sync_copy(out_v, out_hbm.at[pl.ds(base, b_per_w)])

    return gather_kernel


def kernel(values, t, shape):
    batch = t.shape[0]
    ndim = shape.shape[0]
    table_padded = (values.shape[0] + _L - 1) // _L * _L
    values_p = jnp.pad(values, (0, table_padded - values.shape[0]))
    out = _make_gather(batch, table_padded)(values_p, t)
    return out.reshape((batch,) + (1,) * (ndim - 1))


# speedup vs baseline: 4.6531x; 1.0060x over previous
import functools

import jax
import jax.numpy as jnp
from jax import lax
from jax.experimental import pallas as pl
from jax.experimental.pallas import tpu as pltpu
from jax.experimental.pallas import tpu_sc as plsc

_NC = 2
_NS = 16
_NW = _NC * _NS
_L = 16


@functools.lru_cache(maxsize=None)
def _make_gather(batch: int, table_padded: int):
    assert batch % (_NW * _L) == 0
    b_per_w = batch // _NW

    mesh = plsc.VectorSubcoreMesh(core_axis_name="c", subcore_axis_name="s")

    @functools.partial(
        pl.kernel,
        out_type=jax.ShapeDtypeStruct((batch,), jnp.float32),
        mesh=mesh,
        scratch_types=[
            pltpu.VMEM((table_padded,), jnp.float32),
            pltpu.VMEM((b_per_w,), jnp.int32),
            pltpu.VMEM((b_per_w,), jnp.float32),
            pltpu.SemaphoreType.DMA,
            pltpu.SemaphoreType.DMA,
        ],
        compiler_params=pltpu.CompilerParams(needs_layout_passes=False),
    )
    def gather_kernel(values_hbm, t_hbm, out_hbm, tab_v, idx_v, out_v, sem_t, sem_i):
        wid = lax.axis_index("s") * _NC + lax.axis_index("c")
        base = wid * b_per_w
        copy_tab = pltpu.async_copy(values_hbm, tab_v, sem_t)
        copy_idx = pltpu.async_copy(t_hbm.at[pl.ds(base, b_per_w)], idx_v, sem_i)
        copy_tab.wait()
        copy_idx.wait()
        for i in range(b_per_w // _L):
            off = i * _L
            idx16 = idx_v[pl.ds(off, _L)]
            out_v[pl.ds(off, _L)] = plsc.load_gather(tab_v, [idx16])
        pltpu.sync_copy(out_v, out_hbm.at[pl.ds(base, b_per_w)])

    return gather_kernel


def kernel(values, t, shape):
    batch = t.shape[0]
    ndim = shape.shape[0]
    table_padded = (values.shape[0] + _L - 1) // _L * _L
    values_p = jnp.pad(values, (0, table_padded - values.shape[0]))
    out = _make_gather(batch, table_padded)(values_p, t)
    return out.reshape((batch,) + (1,) * (ndim - 1))


# speedup vs baseline: 5.0811x; 1.0920x over previous
import functools

import jax
import jax.numpy as jnp
from jax import lax
from jax.experimental import pallas as pl
from jax.experimental.pallas import tpu as pltpu
from jax.experimental.pallas import tpu_sc as plsc

_NC = 1
_NS = 16
_NW = _NC * _NS
_L = 16


@functools.lru_cache(maxsize=None)
def _make_gather(batch: int, table_padded: int):
    assert batch % (_NW * _L) == 0
    b_per_w = batch // _NW

    mesh = plsc.VectorSubcoreMesh(
        core_axis_name="c", subcore_axis_name="s", num_cores=1
    )

    @functools.partial(
        pl.kernel,
        out_type=jax.ShapeDtypeStruct((batch,), jnp.float32),
        mesh=mesh,
        scratch_types=[
            pltpu.VMEM((table_padded,), jnp.float32),
            pltpu.VMEM((b_per_w,), jnp.int32),
            pltpu.VMEM((b_per_w,), jnp.float32),
            pltpu.SemaphoreType.DMA,
            pltpu.SemaphoreType.DMA,
        ],
        compiler_params=pltpu.CompilerParams(needs_layout_passes=False),
    )
    def gather_kernel(values_hbm, t_hbm, out_hbm, tab_v, idx_v, out_v, sem_t, sem_i):
        wid = lax.axis_index("s") * _NC + lax.axis_index("c")
        base = wid * b_per_w
        copy_tab = pltpu.async_copy(values_hbm, tab_v, sem_t)
        copy_idx = pltpu.async_copy(t_hbm.at[pl.ds(base, b_per_w)], idx_v, sem_i)
        copy_tab.wait()
        copy_idx.wait()
        for i in range(b_per_w // _L):
            off = i * _L
            idx16 = idx_v[pl.ds(off, _L)]
            out_v[pl.ds(off, _L)] = plsc.load_gather(tab_v, [idx16])
        pltpu.sync_copy(out_v, out_hbm.at[pl.ds(base, b_per_w)])

    return gather_kernel


def kernel(values, t, shape):
    batch = t.shape[0]
    ndim = shape.shape[0]
    table_padded = (values.shape[0] + _L - 1) // _L * _L
    values_p = jnp.pad(values, (0, table_padded - values.shape[0]))
    out = _make_gather(batch, table_padded)(values_p, t)
    return out.reshape((batch,) + (1,) * (ndim - 1))
